# Initial kernel scaffold; baseline (speedup 1.0000x reference)
import jax
import jax.numpy as jnp
from jax.experimental import pallas as pl


def kernel(coordinate_start, h, w, support_resolution_h, support_resolution_w, grid):
    raise NotImplementedError("write your pallas kernel here")


# speedup vs baseline: 5.2867x; 5.2867x over previous
import functools

import jax
import jax.numpy as jnp
from jax import lax
from jax.experimental import pallas as pl
from jax.experimental.pallas import tpu as pltpu
from jax.experimental.pallas import tpu_sc as plsc

_C = 96
_G = 256
_B = 2
_NW = 32
_PER_W = (_B * _C) // _NW
_FC = 64
_GR = 72
_EC = 32


def _body(cs_hbm, g_hbm, out_hbm, cs_v, gbuf, fbuf, bufa, bufb, bufc, bufd):
    wid = lax.axis_index("s") * 2 + lax.axis_index("c")
    pltpu.sync_copy(cs_hbm, cs_v)
    lanes = lax.broadcasted_iota(jnp.int32, (16,), 0)
    lanes256 = lanes * _G
    cs_vec = cs_v[...]
    s00, s01, s10, s11 = cs_vec[0], cs_vec[1], cs_vec[2], cs_vec[3]

    def pair_body(q, _):
        pair = wid * _PER_W + q
        b = pair // _C
        c = pair - b * _C
        s0 = jnp.where(b == 0, s00, s10)
        s1 = jnp.where(b == 0, s01, s11)

        def f_chunk(jc, _):
            j0 = jc * _FC
            lo = jnp.minimum((jnp.maximum(s1 + j0 - 1, 0) // 8) * 8, _G - _GR)
            pltpu.sync_copy(g_hbm.at[c, pl.ds(lo, _GR), :], gbuf)

            def f_row(jj, _):
                j = j0 + jj
                u = s1 + j
                beta = jnp.minimum(u, 256).astype(jnp.float32) * (1.0 / 256.0)
                r1 = jnp.clip(u - 1, 0, _G - 1) - lo
                r2 = jnp.clip(u, 0, _G - 1) - lo
                bv = jnp.full((16,), beta, jnp.float32)
                bw = 1.0 - bv
                base = j * _G
                for v in range(16):
                    ga = gbuf[r1, pl.ds(v * 16, 16)]
                    gb = gbuf[r2, pl.ds(v * 16, 16)]
                    fbuf[pl.ds(base + v * 16, 16)] = bv * ga + bw * gb
                return 0

            lax.fori_loop(0, _FC, f_row, 0)
            return 0

        lax.fori_loop(0, 4, f_chunk, 0)
        for v in range(16):
            fbuf[pl.ds(256 * _G + v * 16, 16)] = gbuf[_GR - 1, pl.ds(v * 16, 16)]

        def e_chunk(ec, _):
            i0 = ec * _EC

            def e_row(ii, _):
                i = i0 + ii
                t = s0 + i
                alpha = jnp.minimum(t, 256).astype(jnp.float32) * (1.0 / 256.0)
                av = jnp.full((16,), alpha, jnp.float32)
                aw = 1.0 - av
                ca = jnp.clip(t - 1, 0, _G - 1)
                cb = jnp.clip(t, 0, _G - 1)
                cav = jnp.full((16,), ca, jnp.int32)
                cbv = jnp.full((16,), cb, jnp.int32)
                iim1 = jnp.maximum(ii - 1, 0)
                for v in range(16):
                    basea = lanes256 + (v * 16 * _G)
                    baseb = basea + _G
                    xa = plsc.load_gather(fbuf, [basea + cav])
                    xb = plsc.load_gather(fbuf, [basea + cbv])
                    ra = av * xa + aw * xb
                    bufa[ii, pl.ds(v * 16, 16)] = ra
                    bufb[iim1, pl.ds(v * 16, 16)] = ra
                    ya = plsc.load_gather(fbuf, [baseb + cav])
                    yb = plsc.load_gather(fbuf, [baseb + cbv])
                    rb = av * ya + aw * yb
                    bufc[ii, pl.ds(v * 16, 16)] = rb
                    bufd[iim1, pl.ds(v * 16, 16)] = rb
                return 0

            lax.fori_loop(0, _EC + 1, e_row, 0)
            for k, buf in enumerate([bufa, bufb, bufc, bufd]):
                pltpu.sync_copy(
                    buf.at[pl.ds(0, _EC), :],
                    out_hbm.at[b, k * _C + c, pl.ds(i0, _EC), :],
                )
            return 0

        lax.fori_loop(0, _G // _EC, e_chunk, 0)
        return 0

    lax.fori_loop(0, _PER_W, pair_body, 0)


@functools.partial(jax.jit, static_argnums=())
def _run(cs_pad, g2):
    mesh = plsc.VectorSubcoreMesh(core_axis_name="c", subcore_axis_name="s",
                                  num_cores=2, num_subcores=16)
    fn = pl.kernel(
        _body,
        out_type=jax.ShapeDtypeStruct((_B, 4 * _C, _G, _G), jnp.float32),
        mesh=mesh,
        scratch_types=[
            pltpu.VMEM((16,), jnp.int32),
            pltpu.VMEM((_GR, _G), jnp.float32),
            pltpu.VMEM((257 * _G,), jnp.float32),
            pltpu.VMEM((_EC + 1, _G), jnp.float32),
            pltpu.VMEM((_EC + 1, _G), jnp.float32),
            pltpu.VMEM((_EC + 1, _G), jnp.float32),
            pltpu.VMEM((_EC + 1, _G), jnp.float32),
        ],
        compiler_params=pltpu.CompilerParams(needs_layout_passes=False),
    )
    return fn(cs_pad, g2)


def kernel(coordinate_start, h, w, support_resolution_h, support_resolution_w, grid):
    del h, w, support_resolution_h, support_resolution_w
    cs_pad = jnp.zeros((16,), jnp.int32).at[0:4].set(coordinate_start.reshape(4))
    g2 = grid.reshape(_C, _G, _G)
    return _run(cs_pad, g2)


# speedup vs baseline: 11.3200x; 2.1412x over previous
import functools

import jax
import jax.numpy as jnp
from jax import lax
from jax.experimental import pallas as pl
from jax.experimental.pallas import tpu as pltpu
from jax.experimental.pallas import tpu_sc as plsc

_C = 96
_G = 256
_B = 2
_NW = 32
_PER_W = (_B * _C) // _NW
_FC = 64
_GR = 72
_EC = 32
_FS = 257


def _body(cs_hbm, g_hbm, out_hbm, cs_v, gbuf, fbuf, bufa, bufb, bufc, bufd):
    wid = lax.axis_index("s") * 2 + lax.axis_index("c")
    pltpu.sync_copy(cs_hbm, cs_v)
    lanes = lax.broadcasted_iota(jnp.int32, (16,), 0)
    lanes_fs = lanes * _FS
    cs_vec = cs_v[...]
    s00, s01, s10, s11 = cs_vec[0], cs_vec[1], cs_vec[2], cs_vec[3]

    def pair_body(q, _):
        pair = wid * _PER_W + q
        b = pair // _C
        c = pair - b * _C
        s0 = jnp.where(b == 0, s00, s10)
        s1 = jnp.where(b == 0, s01, s11)

        def f_chunk(jc, _):
            j0 = jc * _FC
            lo = jnp.minimum((jnp.maximum(s1 + j0 - 1, 0) // 8) * 8, _G - _GR)
            pltpu.sync_copy(g_hbm.at[c, pl.ds(lo, _GR), :], gbuf)

            def f_row(jj, _):
                j = j0 + jj
                u = s1 + j
                beta = jnp.minimum(u, 256).astype(jnp.float32) * (1.0 / 256.0)
                r1 = jnp.clip(u - 1, 0, _G - 1) - lo
                r2 = jnp.clip(u, 0, _G - 1) - lo
                bv = jnp.full((16,), beta, jnp.float32)
                bw = 1.0 - bv
                base = j * _FS
                for v in range(16):
                    ga = gbuf[r1, pl.ds(v * 16, 16)]
                    gb = gbuf[r2, pl.ds(v * 16, 16)]
                    fbuf[pl.ds(base + v * 16, 16)] = bv * ga + bw * gb
                return 0

            lax.fori_loop(0, _FC, f_row, 0)
            return 0

        lax.fori_loop(0, 4, f_chunk, 0)
        for v in range(16):
            fbuf[pl.ds(256 * _FS + v * 16, 16)] = gbuf[_GR - 1, pl.ds(v * 16, 16)]

        def e_chunk(ec, _):
            i0 = ec * _EC

            def e_row(ii, _):
                i = i0 + ii
                t = s0 + i
                alpha = jnp.minimum(t, 256).astype(jnp.float32) * (1.0 / 256.0)
                av = jnp.full((16,), alpha, jnp.float32)
                aw = 1.0 - av
                ca = jnp.clip(t - 1, 0, _G - 1)
                cb = jnp.clip(t, 0, _G - 1)
                cav = jnp.full((16,), ca, jnp.int32)
                cbv = jnp.full((16,), cb, jnp.int32)
                iim1 = jnp.maximum(ii - 1, 0)
                for v in range(16):
                    basea = lanes_fs + (v * 16 * _FS)
                    baseb = basea + _FS
                    xa = plsc.load_gather(fbuf, [basea + cav])
                    xb = plsc.load_gather(fbuf, [basea + cbv])
                    ra = av * xa + aw * xb
                    bufa[ii, pl.ds(v * 16, 16)] = ra
                    bufb[iim1, pl.ds(v * 16, 16)] = ra
                    ya = plsc.load_gather(fbuf, [baseb + cav])
                    yb = plsc.load_gather(fbuf, [baseb + cbv])
                    rb = av * ya + aw * yb
                    bufc[ii, pl.ds(v * 16, 16)] = rb
                    bufd[iim1, pl.ds(v * 16, 16)] = rb
                return 0

            lax.fori_loop(0, _EC + 1, e_row, 0)
            for k, buf in enumerate([bufa, bufb, bufc, bufd]):
                pltpu.sync_copy(
                    buf.at[pl.ds(0, _EC), :],
                    out_hbm.at[b, k * _C + c, pl.ds(i0, _EC), :],
                )
            return 0

        lax.fori_loop(0, _G // _EC, e_chunk, 0)
        return 0

    lax.fori_loop(0, _PER_W, pair_body, 0)


@functools.partial(jax.jit, static_argnums=())
def _run(cs_pad, g2):
    mesh = plsc.VectorSubcoreMesh(core_axis_name="c", subcore_axis_name="s",
                                  num_cores=2, num_subcores=16)
    fn = pl.kernel(
        _body,
        out_type=jax.ShapeDtypeStruct((_B, 4 * _C, _G, _G), jnp.float32),
        mesh=mesh,
        scratch_types=[
            pltpu.VMEM((16,), jnp.int32),
            pltpu.VMEM((_GR, _G), jnp.float32),
            pltpu.VMEM((257 * _FS,), jnp.float32),
            pltpu.VMEM((_EC + 1, _G), jnp.float32),
            pltpu.VMEM((_EC + 1, _G), jnp.float32),
            pltpu.VMEM((_EC + 1, _G), jnp.float32),
            pltpu.VMEM((_EC + 1, _G), jnp.float32),
        ],
        compiler_params=pltpu.CompilerParams(needs_layout_passes=False),
    )
    return fn(cs_pad, g2)


def kernel(coordinate_start, h, w, support_resolution_h, support_resolution_w, grid):
    del h, w, support_resolution_h, support_resolution_w
    cs_pad = jnp.zeros((16,), jnp.int32).at[0:4].set(coordinate_start.reshape(4))
    g2 = grid.reshape(_C, _G, _G)
    return _run(cs_pad, g2)


# speedup vs baseline: 11.3983x; 1.0069x over previous
import functools

import jax
import jax.numpy as jnp
from jax import lax
from jax.experimental import pallas as pl
from jax.experimental.pallas import tpu as pltpu
from jax.experimental.pallas import tpu_sc as plsc

_C = 96
_G = 256
_B = 2
_NW = 32
_PER_W = (_B * _C) // _NW
_FC = 64
_GR = 72
_EC = 32
_FS = 257


def _body(cs_hbm, g_hbm, out_hbm, cs_v, gbuf, fbuf, bufa, bufb, bufc, bufd, prev):
    wid = lax.axis_index("s") * 2 + lax.axis_index("c")
    pltpu.sync_copy(cs_hbm, cs_v)
    lanes = lax.broadcasted_iota(jnp.int32, (16,), 0)
    lanes_fs = lanes * _FS
    cs_vec = cs_v[...]
    s00, s01, s10, s11 = cs_vec[0], cs_vec[1], cs_vec[2], cs_vec[3]

    def pair_body(q, _):
        pair = wid * _PER_W + q
        b = pair // _C
        c = pair - b * _C
        s0 = jnp.where(b == 0, s00, s10)
        s1 = jnp.where(b == 0, s01, s11)

        def f_chunk(jc, _):
            j0 = jc * _FC
            lo = jnp.minimum((jnp.maximum(s1 + j0 - 1, 0) // 8) * 8, _G - _GR)
            pltpu.sync_copy(g_hbm.at[c, pl.ds(lo, _GR), :], gbuf)

            def f_row(jj, _):
                j = j0 + jj
                u = s1 + j
                beta = jnp.minimum(u, 256).astype(jnp.float32) * (1.0 / 256.0)
                r1 = jnp.clip(u - 1, 0, _G - 1) - lo
                r2 = jnp.clip(u, 0, _G - 1) - lo
                bv = jnp.full((16,), beta, jnp.float32)
                bw = 1.0 - bv
                base = j * _FS
                for v in range(16):
                    ga = gbuf[r1, pl.ds(v * 16, 16)]
                    gb = gbuf[r2, pl.ds(v * 16, 16)]
                    fbuf[pl.ds(base + v * 16, 16)] = bv * ga + bw * gb
                return 0

            lax.fori_loop(0, _FC, f_row, 0)
            return 0

        lax.fori_loop(0, 4, f_chunk, 0)
        for v in range(16):
            fbuf[pl.ds(256 * _FS + v * 16, 16)] = gbuf[_GR - 1, pl.ds(v * 16, 16)]

        def e_chunk(ec, _):
            i0 = ec * _EC

            t0 = s0 + i0
            alpha0 = jnp.minimum(t0, 256).astype(jnp.float32) * (1.0 / 256.0)
            av0 = jnp.full((16,), alpha0, jnp.float32)
            aw0 = 1.0 - av0
            cav0 = jnp.full((16,), jnp.clip(t0 - 1, 0, _G - 1), jnp.int32)
            cbv0 = jnp.full((16,), jnp.clip(t0, 0, _G - 1), jnp.int32)
            for v in range(16):
                basea = lanes_fs + (v * 16 * _FS)
                baseb = basea + _FS
                xa = plsc.load_gather(fbuf, [basea + cav0])
                xb = plsc.load_gather(fbuf, [basea + cbv0])
                prev[0, pl.ds(v * 16, 16)] = xb
                bufa[0, pl.ds(v * 16, 16)] = av0 * xa + aw0 * xb
                ya = plsc.load_gather(fbuf, [baseb + cav0])
                yb = plsc.load_gather(fbuf, [baseb + cbv0])
                prev[1, pl.ds(v * 16, 16)] = yb
                bufc[0, pl.ds(v * 16, 16)] = av0 * ya + aw0 * yb

            def e_row(ii, _):
                i = i0 + ii
                t = s0 + i
                alpha = jnp.minimum(t, 256).astype(jnp.float32) * (1.0 / 256.0)
                av = jnp.full((16,), alpha, jnp.float32)
                aw = 1.0 - av
                cbv = jnp.full((16,), jnp.clip(t, 0, _G - 1), jnp.int32)
                for v in range(16):
                    basea = lanes_fs + (v * 16 * _FS)
                    baseb = basea + _FS
                    xa = prev[0, pl.ds(v * 16, 16)]
                    xb = plsc.load_gather(fbuf, [basea + cbv])
                    prev[0, pl.ds(v * 16, 16)] = xb
                    ra = av * xa + aw * xb
                    bufa[ii, pl.ds(v * 16, 16)] = ra
                    bufb[ii - 1, pl.ds(v * 16, 16)] = ra
                    ya = prev[1, pl.ds(v * 16, 16)]
                    yb = plsc.load_gather(fbuf, [baseb + cbv])
                    prev[1, pl.ds(v * 16, 16)] = yb
                    rb = av * ya + aw * yb
                    bufc[ii, pl.ds(v * 16, 16)] = rb
                    bufd[ii - 1, pl.ds(v * 16, 16)] = rb
                return 0

            lax.fori_loop(1, _EC + 1, e_row, 0)
            for k, buf in enumerate([bufa, bufb, bufc, bufd]):
                pltpu.sync_copy(
                    buf.at[pl.ds(0, _EC), :],
                    out_hbm.at[b, k * _C + c, pl.ds(i0, _EC), :],
                )
            return 0

        lax.fori_loop(0, _G // _EC, e_chunk, 0)
        return 0

    lax.fori_loop(0, _PER_W, pair_body, 0)


@functools.partial(jax.jit, static_argnums=())
def _run(cs_pad, g2):
    mesh = plsc.VectorSubcoreMesh(core_axis_name="c", subcore_axis_name="s",
                                  num_cores=2, num_subcores=16)
    fn = pl.kernel(
        _body,
        out_type=jax.ShapeDtypeStruct((_B, 4 * _C, _G, _G), jnp.float32),
        mesh=mesh,
        scratch_types=[
            pltpu.VMEM((16,), jnp.int32),
            pltpu.VMEM((_GR, _G), jnp.float32),
            pltpu.VMEM((257 * _FS,), jnp.float32),
            pltpu.VMEM((_EC + 1, _G), jnp.float32),
            pltpu.VMEM((_EC + 1, _G), jnp.float32),
            pltpu.VMEM((_EC + 1, _G), jnp.float32),
            pltpu.VMEM((_EC + 1, _G), jnp.float32),
            pltpu.VMEM((2, _G), jnp.float32),
        ],
        compiler_params=pltpu.CompilerParams(needs_layout_passes=False),
    )
    return fn(cs_pad, g2)


def kernel(coordinate_start, h, w, support_resolution_h, support_resolution_w, grid):
    del h, w, support_resolution_h, support_resolution_w
    cs_pad = jnp.zeros((16,), jnp.int32).at[0:4].set(coordinate_start.reshape(4))
    g2 = grid.reshape(_C, _G, _G)
    return _run(cs_pad, g2)
